# baseline (speedup 1.0000x reference)
import jax
import jax.numpy as jnp
from jax.experimental import pallas as pl
from jax.experimental.pallas import tpu as pltpu


def _stitch_kernel(eid_ref, x_ref, sW_hbm, sb_ref, pW_hbm, pb_ref, o_ref,
                   sW_vmem, pW_vmem, sems):
    i = pl.program_id(0)

    @pl.when(i == 0)
    def _load_weights():
        pltpu.make_async_copy(sW_hbm, sW_vmem, sems.at[0]).start()
        pltpu.make_async_copy(pW_hbm, pW_vmem, sems.at[1]).start()
        pltpu.make_async_copy(sW_hbm, sW_vmem, sems.at[0]).wait()
        pltpu.make_async_copy(pW_hbm, pW_vmem, sems.at[1]).wait()

    e = eid_ref[i]
    x = x_ref[0]
    h = jnp.dot(x, sW_vmem[e], preferred_element_type=jnp.float32)
    h = h + sb_ref[e]
    h = h / (1.0 + jnp.abs(h))
    o = jnp.dot(h, pW_vmem[e], preferred_element_type=jnp.float32)
    o_ref[0] = o + pb_ref[e]


def kernel(x, eid, stitch_W, stitch_b, proj_W, proj_b):
    B, F, N = x.shape
    E, _, M = stitch_W.shape
    P = proj_W.shape[-1]

    eid32 = eid.astype(jnp.int32)
    sb3 = stitch_b.reshape(E, 1, M)
    pb3 = proj_b.reshape(E, 1, P)

    grid_spec = pltpu.PrefetchScalarGridSpec(
        num_scalar_prefetch=1,
        grid=(B,),
        in_specs=[
            pl.BlockSpec((1, F, N), lambda i, eid: (i, 0, 0)),
            pl.BlockSpec(memory_space=pltpu.HBM),
            pl.BlockSpec((E, 1, M), lambda i, eid: (0, 0, 0)),
            pl.BlockSpec(memory_space=pltpu.HBM),
            pl.BlockSpec((E, 1, P), lambda i, eid: (0, 0, 0)),
        ],
        out_specs=pl.BlockSpec((1, F, P), lambda i, eid: (i, 0, 0)),
        scratch_shapes=[
            pltpu.VMEM((E, N, M), jnp.float32),
            pltpu.VMEM((E, M, P), jnp.float32),
            pltpu.SemaphoreType.DMA((2,)),
        ],
    )
    return pl.pallas_call(
        _stitch_kernel,
        grid_spec=grid_spec,
        out_shape=jax.ShapeDtypeStruct((B, F, P), jnp.float32),
    )(eid32, x, stitch_W, sb3, proj_W, pb3)
